# baseline (speedup 1.0000x reference)
import jax
import jax.numpy as jnp
from jax import lax
from jax.experimental import pallas as pl
from jax.experimental.pallas import tpu as pltpu
from jax.experimental.pallas import tpu_sc as plsc

N = 10000
E = 320000
D = 128

NC = 2
NS = 16
NW = NC * NS
EPW = E // NW
K = 128
NCH = 80
EPWP = NCH * K
EPAD = NW * EPWP
NP = 10112
RPT = NP // NS

_mesh = plsc.VectorSubcoreMesh(core_axis_name="c", subcore_axis_name="s")


def _sc_deg_body(ids_hbm, degp_hbm, ia, ib, ones, zb, acc, semia, semib):
    c = lax.axis_index("c")
    s = lax.axis_index("s")
    wid = c * NS + s

    def _fill_z(i, _):
        zb[i] = jnp.zeros((16,), jnp.float32)
        return 0

    lax.fori_loop(0, 16, _fill_z, 0)

    def _fill_1(i, _):
        ones[i] = jnp.ones((16,), jnp.float32)
        return 0

    lax.fori_loop(0, K, _fill_1, 0)
    for q in range(RPT // 16):
        pltpu.sync_copy(zb, acc.at[pl.ds(s * RPT + q * 16, 16)])
    pltpu.sync_copy(zb.at[pl.ds(0, 8)],
                    acc.at[pl.ds(s * RPT + (RPT // 16) * 16, 8)])
    plsc.subcore_barrier()

    def _ldidx(j, buf, sem):
        pltpu.async_copy(ids_hbm.at[wid, j], buf, sem)

    def _widx(buf, sem):
        pltpu.make_async_copy(ids_hbm.at[wid, 0], buf, sem).wait()

    def _scat(buf):
        pltpu.sync_copy(ones, acc.at[buf.at[1]], add=True)

    _ldidx(0, ia, semia)
    _ldidx(1, ib, semib)

    def _pair(t, _):
        j0 = 2 * t
        _widx(ia, semia)
        _scat(ia)
        _ldidx(j0 + 2, ia, semia)
        _widx(ib, semib)
        _scat(ib)
        _ldidx(j0 + 3, ib, semib)
        return 0

    lax.fori_loop(0, NCH // 2 - 1, _pair, 0)
    _widx(ia, semia)
    _scat(ia)
    _widx(ib, semib)
    _scat(ib)
    plsc.subcore_barrier()
    pltpu.sync_copy(acc.at[pl.ds(s * RPT, RPT)],
                    degp_hbm.at[c, pl.ds(s * RPT, RPT)])


def _sc_deg(ids):
    return pl.kernel(
        _sc_deg_body,
        out_type=jax.ShapeDtypeStruct((NC, NP, 16), jnp.float32),
        mesh=_mesh,
        scratch_types=[
            pltpu.VMEM((2, K), jnp.int32),
            pltpu.VMEM((2, K), jnp.int32),
            pltpu.VMEM((K, 16), jnp.float32),
            pltpu.VMEM((16, 16), jnp.float32),
            pltpu.VMEM_SHARED((NP, 16), jnp.float32),
            pltpu.SemaphoreType.DMA,
            pltpu.SemaphoreType.DMA,
        ],
    )(ids)


def _sc_prop_body(g_hbm, ids_hbm, outp_hbm,
                  i0, i1, i2, i3, bufa, bufb, acc,
                  sema, semb, sem0, sem1, sem2, sem3):
    c = lax.axis_index("c")
    s = lax.axis_index("s")
    wid = c * NS + s

    def _fill(i, _):
        for k in range(D // 16):
            bufa[i, pl.ds(k * 16, 16)] = jnp.zeros((16,), jnp.float32)
        return 0

    lax.fori_loop(0, K, _fill, 0)
    for q in range(4):
        pltpu.sync_copy(bufa, acc.at[pl.ds(s * RPT + q * 128, 128)])
    pltpu.sync_copy(bufa.at[pl.ds(0, 120)],
                    acc.at[pl.ds(s * RPT + 512, 120)])
    plsc.subcore_barrier()

    def _ldidx(j, buf, sem):
        pltpu.async_copy(ids_hbm.at[wid, j], buf, sem)

    def _widx(buf, sem):
        pltpu.make_async_copy(ids_hbm.at[wid, 0], buf, sem).wait()

    def _issue(ibuf, buf, sem):
        pltpu.async_copy(g_hbm.at[ibuf.at[0]], buf, sem)

    def _wait(ibuf, buf, sem):
        pltpu.make_async_copy(g_hbm.at[ibuf.at[0]], buf, sem).wait()

    def _scat(ibuf, buf):
        pltpu.sync_copy(buf, acc.at[ibuf.at[1]], add=True)

    _ldidx(0, i0, sem0)
    _ldidx(1, i1, sem1)
    _widx(i0, sem0)
    _issue(i0, bufa, sema)
    _widx(i1, sem1)
    _issue(i1, bufb, semb)
    _ldidx(2, i2, sem2)
    _ldidx(3, i3, sem3)

    def _quad(t, _):
        j0 = 4 * t
        _wait(i0, bufa, sema)
        _scat(i0, bufa)
        _ldidx(j0 + 4, i0, sem0)
        _widx(i2, sem2)
        _issue(i2, bufa, sema)
        _wait(i1, bufb, semb)
        _scat(i1, bufb)
        _ldidx(j0 + 5, i1, sem1)
        _widx(i3, sem3)
        _issue(i3, bufb, semb)
        _wait(i2, bufa, sema)
        _scat(i2, bufa)
        _ldidx(j0 + 6, i2, sem2)
        _widx(i0, sem0)
        _issue(i0, bufa, sema)
        _wait(i3, bufb, semb)
        _scat(i3, bufb)
        _ldidx(j0 + 7, i3, sem3)
        _widx(i1, sem1)
        _issue(i1, bufb, semb)
        return 0

    lax.fori_loop(0, NCH // 4 - 1, _quad, 0)
    _wait(i0, bufa, sema)
    _scat(i0, bufa)
    _widx(i2, sem2)
    _issue(i2, bufa, sema)
    _wait(i1, bufb, semb)
    _scat(i1, bufb)
    _widx(i3, sem3)
    _issue(i3, bufb, semb)
    _wait(i2, bufa, sema)
    _scat(i2, bufa)
    _wait(i3, bufb, semb)
    _scat(i3, bufb)

    plsc.subcore_barrier()
    pltpu.sync_copy(acc.at[pl.ds(s * RPT, RPT)],
                    outp_hbm.at[c, pl.ds(s * RPT, RPT)])


def _sc_prop(g, ids):
    return pl.kernel(
        _sc_prop_body,
        out_type=jax.ShapeDtypeStruct((NC, NP, D), jnp.float32),
        mesh=_mesh,
        scratch_types=[
            pltpu.VMEM((2, K), jnp.int32),
            pltpu.VMEM((2, K), jnp.int32),
            pltpu.VMEM((2, K), jnp.int32),
            pltpu.VMEM((2, K), jnp.int32),
            pltpu.VMEM((K, D), jnp.float32),
            pltpu.VMEM((K, D), jnp.float32),
            pltpu.VMEM_SHARED((NP, D), jnp.float32),
            pltpu.SemaphoreType.DMA,
            pltpu.SemaphoreType.DMA,
            pltpu.SemaphoreType.DMA,
            pltpu.SemaphoreType.DMA,
            pltpu.SemaphoreType.DMA,
            pltpu.SemaphoreType.DMA,
        ],
    )(g, ids)


_TCR = 1000


def _dinv_of(degp_blk):
    deg = degp_blk[0] + degp_blk[1] + 1.0
    return lax.rsqrt(deg)[:, 0:1]


def _tc1_body(degp, x, w, g_out):
    dinv = _dinv_of(degp[...])
    g_out[...] = jnp.dot(x[...], w[...],
                         preferred_element_type=jnp.float32) * dinv


def _tc_mid_body(degp, p, g, b, w, a_out, gn_out):
    dinv = _dinv_of(degp[...])
    pv = p[...]
    a = jnp.maximum((pv[0] + pv[1] + g[...]) * dinv + b[...], 0.0)
    a_out[...] = a
    gn_out[...] = jnp.dot(a, w[...], preferred_element_type=jnp.float32) * dinv


def _tc_fin_body(degp, p, g, b, a0, a1, wjk, bjk, out):
    dinv = _dinv_of(degp[...])
    pv = p[...]
    a2 = jnp.maximum((pv[0] + pv[1] + g[...]) * dinv + b[...], 0.0)
    jk = (a0[...] + a1[...] + a2) * (1.0 / 3.0)
    out[...] = jnp.dot(jk, wjk[...], preferred_element_type=jnp.float32) + bjk[...]


_GRID = N // _TCR
_bs_rows = pl.BlockSpec((_TCR, D), lambda i: (i, 0))
_bs_degp = pl.BlockSpec((NC, _TCR, 16), lambda i: (0, i, 0))
_bs_part = pl.BlockSpec((NC, _TCR, D), lambda i: (0, i, 0))
_bs_w = pl.BlockSpec((D, D), lambda i: (0, 0))
_bs_b = pl.BlockSpec((1, D), lambda i: (0, 0))


def _tc1(degp, x, w):
    return pl.pallas_call(
        _tc1_body,
        grid=(_GRID,),
        in_specs=[_bs_degp, _bs_rows, _bs_w],
        out_specs=_bs_rows,
        out_shape=jax.ShapeDtypeStruct((N, D), jnp.float32),
    )(degp, x, w)


def _tc_mid(degp, p, g, b, w):
    return pl.pallas_call(
        _tc_mid_body,
        grid=(_GRID,),
        in_specs=[_bs_degp, _bs_part, _bs_rows, _bs_b, _bs_w],
        out_specs=[_bs_rows, _bs_rows],
        out_shape=[jax.ShapeDtypeStruct((N, D), jnp.float32),
                   jax.ShapeDtypeStruct((N, D), jnp.float32)],
    )(degp, p, g, b, w)


def _tc_fin(degp, p, g, b, a0, a1, wjk, bjk):
    return pl.pallas_call(
        _tc_fin_body,
        grid=(_GRID,),
        in_specs=[_bs_degp, _bs_part, _bs_rows, _bs_b, _bs_rows, _bs_rows,
                  _bs_w, _bs_b],
        out_specs=_bs_rows,
        out_shape=jax.ShapeDtypeStruct((N, D), jnp.float32),
    )(degp, p, g, b, a0, a1, wjk, bjk)


def kernel(x, edge_index, W0, b0, W1, b1, W2, b2, Wjk, bjk):
    npad = EPAD - E
    pad_src = (jnp.arange(npad, dtype=jnp.int32) % N)
    pad_dst = N + (jnp.arange(npad, dtype=jnp.int32) % (NP - N))
    src = jnp.concatenate([edge_index[0].astype(jnp.int32), pad_src]
                          ).reshape(NW, NCH, 1, K)
    dst = jnp.concatenate([edge_index[1].astype(jnp.int32), pad_dst]
                          ).reshape(NW, NCH, 1, K)
    ids = jnp.concatenate([src, dst], axis=2)
    b0r = b0.reshape(1, D)
    b1r = b1.reshape(1, D)
    b2r = b2.reshape(1, D)
    bjkr = bjk.reshape(1, D)

    degp = _sc_deg(ids)
    g0 = _tc1(degp, x, W0)
    p0 = _sc_prop(g0, ids)
    a0, g1 = _tc_mid(degp, p0, g0, b0r, W1)
    p1 = _sc_prop(g1, ids)
    a1, g2 = _tc_mid(degp, p1, g1, b1r, W2)
    p2 = _sc_prop(g2, ids)
    return _tc_fin(degp, p2, g2, b2r, a0, a1, Wjk, bjkr)


# speedup vs baseline: 1.0402x; 1.0402x over previous
import jax
import jax.numpy as jnp
from jax import lax
from jax.experimental import pallas as pl
from jax.experimental.pallas import tpu as pltpu
from jax.experimental.pallas import tpu_sc as plsc

N = 10000
E = 320000
D = 128

NC = 2
NS = 16
NW = NC * NS
EPW = E // NW
K = 128
NCH = 80
EPWP = NCH * K
EPAD = NW * EPWP
NP = 10112
RPT = NP // NS

_mesh = plsc.VectorSubcoreMesh(core_axis_name="c", subcore_axis_name="s")


def _sc_deg_body(dst4_hbm, degp_hbm, idx, ones, zb, acc):
    c = lax.axis_index("c")
    s = lax.axis_index("s")
    wid = c * NS + s

    pltpu.sync_copy(dst4_hbm.at[wid], idx)

    def _fill_z(i, _):
        zb[i] = jnp.zeros((16,), jnp.float32)
        return 0

    lax.fori_loop(0, 16, _fill_z, 0)

    def _fill_1(i, _):
        ones[i] = jnp.ones((16,), jnp.float32)
        return 0

    lax.fori_loop(0, K, _fill_1, 0)
    for q in range(RPT // 16):
        pltpu.sync_copy(zb, acc.at[pl.ds(s * RPT + q * 16, 16)])
    pltpu.sync_copy(zb.at[pl.ds(0, 8)],
                    acc.at[pl.ds(s * RPT + (RPT // 16) * 16, 8)])
    plsc.subcore_barrier()

    def _chunk(j, _):
        pltpu.sync_copy(ones, acc.at[idx.at[j, 0]], add=True)
        return 0

    lax.fori_loop(0, NCH, _chunk, 0)
    plsc.subcore_barrier()
    pltpu.sync_copy(acc.at[pl.ds(s * RPT, RPT)],
                    degp_hbm.at[c, pl.ds(s * RPT, RPT)])


def _sc_deg(dst4):
    return pl.kernel(
        _sc_deg_body,
        out_type=jax.ShapeDtypeStruct((NC, NP, 16), jnp.float32),
        mesh=_mesh,
        scratch_types=[
            pltpu.VMEM((NCH, 1, K), jnp.int32),
            pltpu.VMEM((K, 16), jnp.float32),
            pltpu.VMEM((16, 16), jnp.float32),
            pltpu.VMEM_SHARED((NP, 16), jnp.float32),
        ],
    )(dst4)


def _sc_prop_body(g_hbm, ids_hbm, outp_hbm,
                  i0, i1, i2, i3, bufa, bufb, acc,
                  sema, semb, sem0, sem1, sem2, sem3):
    c = lax.axis_index("c")
    s = lax.axis_index("s")
    wid = c * NS + s

    def _fill(i, _):
        for k in range(D // 16):
            bufa[i, pl.ds(k * 16, 16)] = jnp.zeros((16,), jnp.float32)
        return 0

    lax.fori_loop(0, K, _fill, 0)
    for q in range(4):
        pltpu.sync_copy(bufa, acc.at[pl.ds(s * RPT + q * 128, 128)])
    pltpu.sync_copy(bufa.at[pl.ds(0, 120)],
                    acc.at[pl.ds(s * RPT + 512, 120)])
    plsc.subcore_barrier()

    def _ldidx(j, buf, sem):
        pltpu.async_copy(ids_hbm.at[wid, j], buf, sem)

    def _widx(j, buf, sem):
        pltpu.make_async_copy(ids_hbm.at[wid, j], buf, sem).wait()

    def _issue(ibuf, buf, sem):
        pltpu.async_copy(g_hbm.at[ibuf.at[0]], buf, sem)

    def _wait(ibuf, buf, sem):
        pltpu.make_async_copy(g_hbm.at[ibuf.at[0]], buf, sem).wait()

    def _scat(ibuf, buf):
        pltpu.sync_copy(buf, acc.at[ibuf.at[1]], add=True)

    _ldidx(0, i0, sem0)
    _ldidx(1, i1, sem1)
    _widx(0, i0, sem0)
    _issue(i0, bufa, sema)
    _widx(1, i1, sem1)
    _issue(i1, bufb, semb)
    _ldidx(2, i2, sem2)
    _ldidx(3, i3, sem3)

    def _quad(t, _):
        j0 = 4 * t
        _wait(i0, bufa, sema)
        _scat(i0, bufa)
        _ldidx(j0 + 4, i0, sem0)
        _widx(j0 + 2, i2, sem2)
        _issue(i2, bufa, sema)
        _wait(i1, bufb, semb)
        _scat(i1, bufb)
        _ldidx(j0 + 5, i1, sem1)
        _widx(j0 + 3, i3, sem3)
        _issue(i3, bufb, semb)
        _wait(i2, bufa, sema)
        _scat(i2, bufa)
        _ldidx(j0 + 6, i2, sem2)
        _widx(j0 + 4, i0, sem0)
        _issue(i0, bufa, sema)
        _wait(i3, bufb, semb)
        _scat(i3, bufb)
        _ldidx(j0 + 7, i3, sem3)
        _widx(j0 + 5, i1, sem1)
        _issue(i1, bufb, semb)
        return 0

    lax.fori_loop(0, NCH // 4 - 1, _quad, 0)
    _wait(i0, bufa, sema)
    _scat(i0, bufa)
    _widx(NCH - 2, i2, sem2)
    _issue(i2, bufa, sema)
    _wait(i1, bufb, semb)
    _scat(i1, bufb)
    _widx(NCH - 1, i3, sem3)
    _issue(i3, bufb, semb)
    _wait(i2, bufa, sema)
    _scat(i2, bufa)
    _wait(i3, bufb, semb)
    _scat(i3, bufb)

    plsc.subcore_barrier()
    pltpu.sync_copy(acc.at[pl.ds(s * RPT, RPT)],
                    outp_hbm.at[c, pl.ds(s * RPT, RPT)])


def _sc_prop(g, ids):
    return pl.kernel(
        _sc_prop_body,
        out_type=jax.ShapeDtypeStruct((NC, NP, D), jnp.float32),
        mesh=_mesh,
        scratch_types=[
            pltpu.VMEM((2, K), jnp.int32),
            pltpu.VMEM((2, K), jnp.int32),
            pltpu.VMEM((2, K), jnp.int32),
            pltpu.VMEM((2, K), jnp.int32),
            pltpu.VMEM((K, D), jnp.float32),
            pltpu.VMEM((K, D), jnp.float32),
            pltpu.VMEM_SHARED((NP, D), jnp.float32),
            pltpu.SemaphoreType.DMA,
            pltpu.SemaphoreType.DMA,
            pltpu.SemaphoreType.DMA,
            pltpu.SemaphoreType.DMA,
            pltpu.SemaphoreType.DMA,
            pltpu.SemaphoreType.DMA,
        ],
    )(g, ids)


_TCR = 1000


def _dinv_of(degp_blk):
    deg = degp_blk[0] + degp_blk[1] + 1.0
    return lax.rsqrt(deg)[:, 0:1]


def _tc1_body(degp, x, w, g_out):
    dinv = _dinv_of(degp[...])
    g_out[...] = jnp.dot(x[...], w[...],
                         preferred_element_type=jnp.float32) * dinv


def _tc_mid_body(degp, p, g, b, w, a_out, gn_out):
    dinv = _dinv_of(degp[...])
    pv = p[...]
    a = jnp.maximum((pv[0] + pv[1] + g[...]) * dinv + b[...], 0.0)
    a_out[...] = a
    gn_out[...] = jnp.dot(a, w[...], preferred_element_type=jnp.float32) * dinv


def _tc_fin_body(degp, p, g, b, a0, a1, wjk, bjk, out):
    dinv = _dinv_of(degp[...])
    pv = p[...]
    a2 = jnp.maximum((pv[0] + pv[1] + g[...]) * dinv + b[...], 0.0)
    jk = (a0[...] + a1[...] + a2) * (1.0 / 3.0)
    out[...] = jnp.dot(jk, wjk[...], preferred_element_type=jnp.float32) + bjk[...]


_GRID = N // _TCR
_bs_rows = pl.BlockSpec((_TCR, D), lambda i: (i, 0))
_bs_degp = pl.BlockSpec((NC, _TCR, 16), lambda i: (0, i, 0))
_bs_part = pl.BlockSpec((NC, _TCR, D), lambda i: (0, i, 0))
_bs_w = pl.BlockSpec((D, D), lambda i: (0, 0))
_bs_b = pl.BlockSpec((1, D), lambda i: (0, 0))


def _tc1(degp, x, w):
    return pl.pallas_call(
        _tc1_body,
        grid=(_GRID,),
        in_specs=[_bs_degp, _bs_rows, _bs_w],
        out_specs=_bs_rows,
        out_shape=jax.ShapeDtypeStruct((N, D), jnp.float32),
    )(degp, x, w)


def _tc_mid(degp, p, g, b, w):
    return pl.pallas_call(
        _tc_mid_body,
        grid=(_GRID,),
        in_specs=[_bs_degp, _bs_part, _bs_rows, _bs_b, _bs_w],
        out_specs=[_bs_rows, _bs_rows],
        out_shape=[jax.ShapeDtypeStruct((N, D), jnp.float32),
                   jax.ShapeDtypeStruct((N, D), jnp.float32)],
    )(degp, p, g, b, w)


def _tc_fin(degp, p, g, b, a0, a1, wjk, bjk):
    return pl.pallas_call(
        _tc_fin_body,
        grid=(_GRID,),
        in_specs=[_bs_degp, _bs_part, _bs_rows, _bs_b, _bs_rows, _bs_rows,
                  _bs_w, _bs_b],
        out_specs=_bs_rows,
        out_shape=jax.ShapeDtypeStruct((N, D), jnp.float32),
    )(degp, p, g, b, a0, a1, wjk, bjk)


def kernel(x, edge_index, W0, b0, W1, b1, W2, b2, Wjk, bjk):
    npad = EPAD - E
    pad_src = (jnp.arange(npad, dtype=jnp.int32) % N)
    pad_dst = N + (jnp.arange(npad, dtype=jnp.int32) % (NP - N))
    src = jnp.concatenate([edge_index[0].astype(jnp.int32), pad_src]
                          ).reshape(NW, NCH, 1, K)
    dst = jnp.concatenate([edge_index[1].astype(jnp.int32), pad_dst]
                          ).reshape(NW, NCH, 1, K)
    ids = jnp.concatenate([src, dst], axis=2)
    dst4 = dst
    b0r = b0.reshape(1, D)
    b1r = b1.reshape(1, D)
    b2r = b2.reshape(1, D)
    bjkr = bjk.reshape(1, D)

    degp = _sc_deg(dst4)
    g0 = _tc1(degp, x, W0)
    p0 = _sc_prop(g0, ids)
    a0, g1 = _tc_mid(degp, p0, g0, b0r, W1)
    p1 = _sc_prop(g1, ids)
    a1, g2 = _tc_mid(degp, p1, g1, b1r, W2)
    p2 = _sc_prop(g2, ids)
    return _tc_fin(degp, p2, g2, b2r, a0, a1, Wjk, bjkr)
